# baseline (speedup 1.0000x reference)
import functools

import jax
import jax.numpy as jnp
from jax import lax
from jax.experimental import pallas as pl
from jax.experimental.pallas import tpu as pltpu
from jax.experimental.pallas import tpu_sc as plsc

N = 10000
E = 160000
D = 128
R = 16
G = 256
H = 3 * D

NC = 2
NS = 16
NW = NC * NS
C = 128
E_PAD = 163840
EW = E_PAD // NW
NCH = EW // C


def _bn_body(x_ref, w_ref, b_ref, o_ref):
    x = x_ref[...]
    mu = jnp.mean(x, axis=0, keepdims=True)
    var = jnp.mean(jnp.square(x - mu), axis=0, keepdims=True)
    o_ref[...] = (x - mu) * jax.lax.rsqrt(var + 1e-5) * w_ref[...] + b_ref[...]


def _batchnorm(x, w, b):
    n, d = x.shape
    return pl.pallas_call(
        _bn_body,
        out_shape=jax.ShapeDtypeStruct((n, d), jnp.float32),
    )(x, w.reshape(1, d), b.reshape(1, d))


def _transform_body(h_ref, w_ref, o_ref):
    o_ref[0, ...] = jax.lax.dot_general(
        h_ref[...], w_ref[0, ...], (((1,), (1,)), ((), ())),
        preferred_element_type=jnp.float32)


def _transform(h, W_r):
    NB = 10
    nb = N // NB
    return pl.pallas_call(
        _transform_body,
        grid=(R, NB),
        in_specs=[
            pl.BlockSpec((nb, D), lambda r, i: (i, 0)),
            pl.BlockSpec((1, D, D), lambda r, i: (r, 0, 0)),
        ],
        out_specs=pl.BlockSpec((1, nb, D), lambda r, i: (r, i, 0)),
        out_shape=jax.ShapeDtypeStruct((R, N, D), jnp.float32),
    )(h, W_r)


def _score_body(hs_ref, hd_ref, rel_ref, emb_ref, s_ref, m_ref):
    rel = rel_ref[0, ...]
    onehot = (rel[:, None] == lax.broadcasted_iota(jnp.int32, (1, R), 1)
              ).astype(jnp.float32)
    e_r = jax.lax.dot_general(onehot, emb_ref[...], (((1,), (0,)), ((), ())),
                              preferred_element_type=jnp.float32, precision=lax.Precision.HIGHEST)
    t = jnp.tanh(hs_ref[...] + e_r)
    s = jnp.sum(hd_ref[...] * t, axis=1)
    s_ref[0, ...] = s
    bm = jnp.max(s)
    i = pl.program_id(0)

    @pl.when(i == 0)
    def _init():
        m_ref[0, 0] = bm

    @pl.when(i > 0)
    def _acc():
        m_ref[0, 0] = jnp.maximum(m_ref[0, 0], bm)


def _scores(h_src, h_dst, rel, rel_emb):
    EB = 25
    eb = E // EB
    s, m = pl.pallas_call(
        _score_body,
        grid=(EB,),
        in_specs=[
            pl.BlockSpec((eb, D), lambda i: (i, 0)),
            pl.BlockSpec((eb, D), lambda i: (i, 0)),
            pl.BlockSpec((1, eb), lambda i: (0, i)),
            pl.BlockSpec((R, D), lambda i: (0, 0)),
        ],
        out_specs=[
            pl.BlockSpec((1, eb), lambda i: (0, i)),
            pl.BlockSpec((1, 1), lambda i: (0, 0), memory_space=pltpu.SMEM),
        ],
        out_shape=[
            jax.ShapeDtypeStruct((1, E), jnp.float32),
            jax.ShapeDtypeStruct((1, 1), jnp.float32),
        ],
    )(h_src, h_dst, rel.reshape(1, E), rel_emb)
    return s.reshape(E), m[0, 0]


def _linear_body(s_ref, d_ref, w_ref, b_ref, o_ref):
    agg = s_ref[...] / jnp.maximum(d_ref[...], 1e-30)
    y = jax.lax.dot_general(agg, w_ref[...], (((1,), (1,)), ((), ())),
                            preferred_element_type=jnp.float32)
    o_ref[...] = jax.nn.relu(y + b_ref[...])


def _linear_relu(S, denom, Wl, bl):
    NB = 10
    nb = N // NB
    denom = denom.reshape(N, 1)
    return pl.pallas_call(
        _linear_body,
        grid=(NB,),
        in_specs=[
            pl.BlockSpec((nb, D), lambda i: (i, 0)),
            pl.BlockSpec((nb, 1), lambda i: (i, 0)),
            pl.BlockSpec((D, D), lambda i: (0, 0)),
            pl.BlockSpec((1, D), lambda i: (0, 0)),
        ],
        out_specs=pl.BlockSpec((nb, D), lambda i: (i, 0)),
        out_shape=jax.ShapeDtypeStruct((N, D), jnp.float32),
    )(S, denom, Wl, bl.reshape(1, D))


def _head_body(cat_ref, batch_ref, w_ref, b_ref, pw1_ref, pb1_ref,
               pw2_ref, pb2_ref, o_ref):
    onehot = (batch_ref[0, :][:, None] ==
              lax.broadcasted_iota(jnp.int32, (1, G), 1)).astype(jnp.float32)
    sums = jax.lax.dot_general(onehot, cat_ref[...], (((0,), (0,)), ((), ())),
                               preferred_element_type=jnp.float32, precision=lax.Precision.HIGHEST)
    counts = jnp.sum(onehot, axis=0)
    pooled = sums / jnp.maximum(counts, 1.0)[:, None]
    mu = jnp.mean(pooled, axis=0, keepdims=True)
    var = jnp.mean(jnp.square(pooled - mu), axis=0, keepdims=True)
    pooled = (pooled - mu) * jax.lax.rsqrt(var + 1e-5) * w_ref[...] + b_ref[...]
    y = jax.nn.relu(
        jax.lax.dot_general(pooled, pw1_ref[...], (((1,), (1,)), ((), ())),
                            preferred_element_type=jnp.float32) + pb1_ref[...])
    o_ref[...] = jax.lax.dot_general(
        y, pw2_ref[...], (((1,), (1,)), ((), ())),
        preferred_element_type=jnp.float32) + pb2_ref[...]


def _pool_head(cat, batch, bnh_w, bnh_b, pW1, pb1, pW2, pb2):
    return pl.pallas_call(
        _head_body,
        out_shape=jax.ShapeDtypeStruct((G, H), jnp.float32),
    )(cat, batch.reshape(1, N), bnh_w.reshape(1, H), bnh_b.reshape(1, H),
      pW1, pb1.reshape(1, H), pW2, pb2.reshape(1, H))


def _sc_gather2(xt, idx_src3, idx_dst3):
    mesh = plsc.VectorSubcoreMesh(core_axis_name="c", subcore_axis_name="s")

    @functools.partial(
        pl.kernel, mesh=mesh,
        out_type=[jax.ShapeDtypeStruct((E_PAD, D), jnp.float32),
                  jax.ShapeDtypeStruct((E_PAD, D), jnp.float32)],
        scratch_types=[
            pltpu.VMEM((NCH, C), jnp.int32),
            pltpu.VMEM((NCH, C), jnp.int32),
            pltpu.VMEM((C, D), jnp.float32),
            pltpu.VMEM((C, D), jnp.float32),
            pltpu.SemaphoreType.DMA,
            pltpu.SemaphoreType.DMA,
        ],
    )
    def k(xt_hbm, is_hbm, id_hbm, os_hbm, od_hbm,
          is_v, id_v, rs_v, rd_v, sem_s, sem_d):
        wid = lax.axis_index("s") * NC + lax.axis_index("c")
        pltpu.sync_copy(is_hbm.at[wid], is_v)
        pltpu.sync_copy(id_hbm.at[wid], id_v)
        base = wid * EW

        def body(j, carry):
            cs = pltpu.async_copy(xt_hbm.at[is_v.at[j]], rs_v, sem_s)
            cd = pltpu.async_copy(xt_hbm.at[id_v.at[j]], rd_v, sem_d)
            cs.wait()
            cd.wait()
            pltpu.sync_copy(rs_v, os_hbm.at[pl.ds(base + j * C, C)])
            pltpu.sync_copy(rd_v, od_hbm.at[pl.ds(base + j * C, C)])
            return carry

        lax.fori_loop(0, NCH, body, 0)

    return k(xt, idx_src3, idx_dst3)


def _aggregate(scores, gmax, src, dst, h):
    smax = jax.ops.segment_max(scores, dst, num_segments=N)
    sexp = jnp.exp(scores - smax[dst])
    denom = jax.ops.segment_sum(sexp, dst, num_segments=N)
    S = jax.ops.segment_sum(sexp[:, None] * h[src], dst, num_segments=N)
    return S, denom


def _cagat_layer(h, src, dst, rel, idx_src3, idx_dst3, W_r, rel_emb, Wl, bl):
    xt = _transform(h, W_r).reshape(R * N, D)
    hs_p, hd_p = _sc_gather2(xt, idx_src3, idx_dst3)
    h_src = hs_p[:E]
    h_dst = hd_p[:E]
    scores, gmax = _scores(h_src, h_dst, rel, rel_emb)
    S, denom = _aggregate(scores, gmax, src, dst, h)
    return _linear_relu(S, denom, Wl, bl)


def kernel(x, edge_index, batch, edge_attr, W_r, relation_embedding,
           bn0_w, bn0_b, bn1_w, bn1_b, Wl0, bl0, Wl1, bl1,
           bnh_w, bnh_b, pW1, pb1, pW2, pb2):
    src = edge_index[0]
    dst = edge_index[1]
    rel = edge_attr

    def pad3(a):
        return jnp.concatenate(
            [a, jnp.zeros((E_PAD - E,), a.dtype)]).reshape(NW, NCH, C)

    idx_src3 = pad3(rel * N + src)
    idx_dst3 = pad3(rel * N + dst)

    h0 = _batchnorm(x, bn0_w, bn0_b)
    h1 = _cagat_layer(h0, src, dst, rel, idx_src3, idx_dst3,
                      W_r, relation_embedding, Wl0, bl0)
    h2in = _batchnorm(h1, bn1_w, bn1_b)
    h2 = _cagat_layer(h2in, src, dst, rel, idx_src3, idx_dst3,
                      W_r, relation_embedding, Wl1, bl1)

    cat = jnp.concatenate([x, h1, h2], axis=1)
    return _pool_head(cat, batch, bnh_w, bnh_b, pW1, pb1, pW2, pb2)


# speedup vs baseline: 2.4678x; 2.4678x over previous
import functools

import jax
import jax.numpy as jnp
from jax import lax
from jax.experimental import pallas as pl
from jax.experimental.pallas import tpu as pltpu
from jax.experimental.pallas import tpu_sc as plsc

N = 10000
E = 160000
D = 128
R = 16
G = 256
H = 3 * D

NC = 2
NS = 16
NW = NC * NS
C = 128
E_PAD = 163840
EW = E_PAD // NW
NCH = EW // C
N_PAD = 10240


def _bn_body(x_ref, w_ref, b_ref, o_ref):
    x = x_ref[...]
    mu = jnp.mean(x, axis=0, keepdims=True)
    var = jnp.mean(jnp.square(x - mu), axis=0, keepdims=True)
    o_ref[...] = (x - mu) * jax.lax.rsqrt(var + 1e-5) * w_ref[...] + b_ref[...]


def _batchnorm(x, w, b):
    n, d = x.shape
    return pl.pallas_call(
        _bn_body,
        out_shape=jax.ShapeDtypeStruct((n, d), jnp.float32),
    )(x, w.reshape(1, d), b.reshape(1, d))


def _transform_body(h_ref, w_ref, o_ref):
    o_ref[0, ...] = jax.lax.dot_general(
        h_ref[...], w_ref[0, ...], (((1,), (1,)), ((), ())),
        preferred_element_type=jnp.float32)


def _transform(h, W_r):
    NB = 10
    nb = N // NB
    return pl.pallas_call(
        _transform_body,
        grid=(R, NB),
        in_specs=[
            pl.BlockSpec((nb, D), lambda r, i: (i, 0)),
            pl.BlockSpec((1, D, D), lambda r, i: (r, 0, 0)),
        ],
        out_specs=pl.BlockSpec((1, nb, D), lambda r, i: (r, i, 0)),
        out_shape=jax.ShapeDtypeStruct((R, N, D), jnp.float32),
    )(h, W_r)


def _score_body(hs_ref, hd_ref, rel_ref, dst_ref, emb_ref,
                s_ref, v_ref, bidx_ref):
    rel = rel_ref[0, ...]
    onehot = (rel[:, None] == lax.broadcasted_iota(jnp.int32, (1, R), 1)
              ).astype(jnp.float32)
    e_r = jax.lax.dot_general(onehot, emb_ref[...], (((1,), (0,)), ((), ())),
                              preferred_element_type=jnp.float32,
                              precision=lax.Precision.HIGHEST)
    t = jnp.tanh(hs_ref[...] + e_r)
    s = jnp.sum(hd_ref[...] * t, axis=1)
    eb = s_ref.shape[1]
    i = pl.program_id(0)
    gidx = i * eb + lax.broadcasted_iota(jnp.int32, (1, eb), 1)[0]
    s = jnp.where(gidx < E, s, -1e30)
    s_ref[0, ...] = s
    k2 = s * 1.4426950408889634
    b = jnp.clip(jnp.floor(k2 * 0.03125), -8.0, 7.0)
    v_ref[0, ...] = jnp.exp2(k2 - 32.0 * b)
    bidx_ref[0, ...] = (b.astype(jnp.int32) + 8) * N + dst_ref[0, ...]


def _scores(h_src, h_dst, rel_p, dst_p, rel_emb):
    EB = 32
    eb = E_PAD // EB
    s, v, bidx = pl.pallas_call(
        _score_body,
        grid=(EB,),
        in_specs=[
            pl.BlockSpec((eb, D), lambda i: (i, 0)),
            pl.BlockSpec((eb, D), lambda i: (i, 0)),
            pl.BlockSpec((1, eb), lambda i: (0, i)),
            pl.BlockSpec((1, eb), lambda i: (0, i)),
            pl.BlockSpec((R, D), lambda i: (0, 0)),
        ],
        out_specs=[
            pl.BlockSpec((1, eb), lambda i: (0, i)),
            pl.BlockSpec((1, eb), lambda i: (0, i)),
            pl.BlockSpec((1, eb), lambda i: (0, i)),
        ],
        out_shape=[
            jax.ShapeDtypeStruct((1, E_PAD), jnp.float32),
            jax.ShapeDtypeStruct((1, E_PAD), jnp.float32),
            jax.ShapeDtypeStruct((1, E_PAD), jnp.int32),
        ],
    )(h_src, h_dst, rel_p.reshape(1, E_PAD), dst_p.reshape(1, E_PAD), rel_emb)
    return (s.reshape(NW, NCH, C), v.reshape(NW, NCH, C),
            bidx.reshape(NW, NCH, C))


NBAND = 16


def _sc_banded_denom(v3, bidx3):
    mesh = plsc.VectorSubcoreMesh(core_axis_name="c", subcore_axis_name="s")

    @functools.partial(
        pl.kernel, mesh=mesh,
        out_type=jax.ShapeDtypeStruct((NC * NBAND * N,), jnp.float32),
        scratch_types=[
            pltpu.VMEM((NCH, C), jnp.float32),
            pltpu.VMEM((NCH, C), jnp.int32),
            pltpu.VMEM((NBAND * N // NS,), jnp.float32),
            pltpu.VMEM_SHARED((NBAND * N,), jnp.float32),
        ],
    )
    def k(v_hbm, b_hbm, out_hbm, v_v, b_v, o_v, acc):
        cid = lax.axis_index("c")
        sid = lax.axis_index("s")
        wid = sid * NC + cid

        chunk = NBAND * N // NS

        def zbody(i, carry):
            o_v[pl.ds(i * 16, 16)] = jnp.zeros((16,), jnp.float32)
            return carry

        lax.fori_loop(0, chunk // 16, zbody, 0)
        pltpu.sync_copy(o_v, acc.at[pl.ds(sid * chunk, chunk)])
        plsc.subcore_barrier()

        pltpu.sync_copy(v_hbm.at[wid], v_v)
        pltpu.sync_copy(b_hbm.at[wid], b_v)

        def body(j, carry):
            pltpu.sync_copy(v_v.at[j], acc.at[b_v.at[j]], add=True)
            return carry

        lax.fori_loop(0, NCH, body, 0)
        plsc.subcore_barrier()
        pltpu.sync_copy(acc.at[pl.ds(sid * chunk, chunk)], o_v)
        pltpu.sync_copy(o_v, out_hbm.at[pl.ds(cid * (NBAND * N) + sid * chunk,
                                              chunk)])

    return k(v3, bidx3)


def _lse_body(a_ref, o_ref):
    at = a_ref[0, ...] + a_ref[1, ...]
    biota = lax.broadcasted_iota(jnp.int32, (NBAND, N), 0).astype(jnp.float32)
    btop = jnp.max(jnp.where(at > 0.0, biota, -1.0), axis=0)
    fac = jnp.exp2(32.0 * jnp.clip(biota - btop[None, :], -2.0, 0.0))
    denom = jnp.sum(at * fac, axis=0)
    lse = jnp.where(btop >= 0.0,
                    0.6931471805599453 * 32.0 * (btop - 8.0)
                    + jnp.log(jnp.maximum(denom, 1e-38)), 0.0)
    o_ref[0, ...] = lse


def _lse(A2):
    return pl.pallas_call(
        _lse_body,
        out_shape=jax.ShapeDtypeStruct((1, N), jnp.float32),
    )(A2.reshape(NC, NBAND, N)).reshape(N)


def _sc_weighted_agg(s3, src3, dst3, lse, h):
    mesh = plsc.VectorSubcoreMesh(core_axis_name="c", subcore_axis_name="s")
    ROWZ = 128

    @functools.partial(
        pl.kernel, mesh=mesh,
        compiler_params=pltpu.CompilerParams(needs_layout_passes=False),
        out_type=jax.ShapeDtypeStruct((NC, N_PAD, D), jnp.float32),
        scratch_types=[
            pltpu.VMEM((C,), jnp.float32),
            pltpu.VMEM((C,), jnp.int32),
            pltpu.VMEM((C,), jnp.int32),
            pltpu.VMEM((N,), jnp.float32),
            pltpu.VMEM((C, D), jnp.float32),
            pltpu.VMEM((C,), jnp.float32),
            pltpu.VMEM_SHARED((N_PAD, D), jnp.float32),
            pltpu.SemaphoreType.DMA,
        ],
    )
    def k(s_hbm, src_hbm, dst_hbm, lse_hbm, h_hbm, out_hbm,
          sc_v, srcc_v, dstc_v, lse_v, rows_v, w_v, acc, sem):
        cid = lax.axis_index("c")
        sid = lax.axis_index("s")
        wid = sid * NC + cid

        def zbody(i, carry):
            for kk in range(D // 16):
                rows_v[i, pl.ds(kk * 16, 16)] = jnp.zeros((16,), jnp.float32)
            return carry

        lax.fori_loop(0, ROWZ, zbody, 0)
        for t in range(5):
            pltpu.sync_copy(
                rows_v, acc.at[pl.ds(sid * (N_PAD // NS) + t * ROWZ, ROWZ)])
        plsc.subcore_barrier()

        pltpu.sync_copy(lse_hbm, lse_v)

        def body(j, carry):
            pltpu.sync_copy(s_hbm.at[wid, j], sc_v)
            pltpu.sync_copy(src_hbm.at[wid, j], srcc_v)
            pltpu.sync_copy(dst_hbm.at[wid, j], dstc_v)
            pltpu.async_copy(h_hbm.at[srcc_v], rows_v, sem).wait()
            for i in range(C // 16):
                dvec = dstc_v[pl.ds(i * 16, 16)]
                lg = plsc.load_gather(lse_v, [dvec])
                w_v[pl.ds(i * 16, 16)] = jnp.exp(sc_v[pl.ds(i * 16, 16)] - lg)

            def scale(e, carry2):
                wb = plsc.load_gather(w_v, [jnp.full((16,), e, jnp.int32)])
                row = rows_v.at[e]
                for kk in range(D // 16):
                    row[pl.ds(kk * 16, 16)] = row[pl.ds(kk * 16, 16)] * wb
                return carry2

            lax.fori_loop(0, C, scale, 0)
            pltpu.sync_copy(rows_v, acc.at[dstc_v], add=True)
            return carry

        lax.fori_loop(0, NCH, body, 0)
        plsc.subcore_barrier()
        for t in range(5):
            r0 = sid * (N_PAD // NS) + t * ROWZ
            pltpu.sync_copy(acc.at[pl.ds(r0, ROWZ)], rows_v)
            pltpu.sync_copy(rows_v, out_hbm.at[cid, pl.ds(r0, ROWZ)])

    return k(s3, src3, dst3, lse, h)


def _linear_body(sa_ref, sb_ref, w_ref, b_ref, o_ref):
    agg = sa_ref[0, ...] + sb_ref[0, ...]
    y = jax.lax.dot_general(agg, w_ref[...], (((1,), (1,)), ((), ())),
                            preferred_element_type=jnp.float32)
    o_ref[...] = jax.nn.relu(y + b_ref[...])


def _linear_relu(S2, Wl, bl):
    NB = 10
    nb = N // NB
    return pl.pallas_call(
        _linear_body,
        grid=(NB,),
        in_specs=[
            pl.BlockSpec((1, nb, D), lambda i: (0, i, 0)),
            pl.BlockSpec((1, nb, D), lambda i: (1, i, 0)),
            pl.BlockSpec((D, D), lambda i: (0, 0)),
            pl.BlockSpec((1, D), lambda i: (0, 0)),
        ],
        out_specs=pl.BlockSpec((nb, D), lambda i: (i, 0)),
        out_shape=jax.ShapeDtypeStruct((N, D), jnp.float32),
    )(S2, S2, Wl, bl.reshape(1, D))


def _head_body(cat_ref, batch_ref, w_ref, b_ref, pw1_ref, pb1_ref,
               pw2_ref, pb2_ref, o_ref):
    onehot = (batch_ref[0, :][:, None] ==
              lax.broadcasted_iota(jnp.int32, (1, G), 1)).astype(jnp.float32)
    sums = jax.lax.dot_general(onehot, cat_ref[...], (((0,), (0,)), ((), ())),
                               preferred_element_type=jnp.float32, precision=lax.Precision.HIGHEST)
    counts = jnp.sum(onehot, axis=0)
    pooled = sums / jnp.maximum(counts, 1.0)[:, None]
    mu = jnp.mean(pooled, axis=0, keepdims=True)
    var = jnp.mean(jnp.square(pooled - mu), axis=0, keepdims=True)
    pooled = (pooled - mu) * jax.lax.rsqrt(var + 1e-5) * w_ref[...] + b_ref[...]
    y = jax.nn.relu(
        jax.lax.dot_general(pooled, pw1_ref[...], (((1,), (1,)), ((), ())),
                            preferred_element_type=jnp.float32) + pb1_ref[...])
    o_ref[...] = jax.lax.dot_general(
        y, pw2_ref[...], (((1,), (1,)), ((), ())),
        preferred_element_type=jnp.float32) + pb2_ref[...]


def _pool_head(cat, batch, bnh_w, bnh_b, pW1, pb1, pW2, pb2):
    return pl.pallas_call(
        _head_body,
        out_shape=jax.ShapeDtypeStruct((G, H), jnp.float32),
    )(cat, batch.reshape(1, N), bnh_w.reshape(1, H), bnh_b.reshape(1, H),
      pW1, pb1.reshape(1, H), pW2, pb2.reshape(1, H))


def _sc_gather2(xt, idx_src3, idx_dst3):
    mesh = plsc.VectorSubcoreMesh(core_axis_name="c", subcore_axis_name="s")

    @functools.partial(
        pl.kernel, mesh=mesh,
        out_type=[jax.ShapeDtypeStruct((E_PAD, D), jnp.float32),
                  jax.ShapeDtypeStruct((E_PAD, D), jnp.float32)],
        scratch_types=[
            pltpu.VMEM((NCH, C), jnp.int32),
            pltpu.VMEM((NCH, C), jnp.int32),
            pltpu.VMEM((C, D), jnp.float32),
            pltpu.VMEM((C, D), jnp.float32),
            pltpu.SemaphoreType.DMA,
            pltpu.SemaphoreType.DMA,
        ],
    )
    def k(xt_hbm, is_hbm, id_hbm, os_hbm, od_hbm,
          is_v, id_v, rs_v, rd_v, sem_s, sem_d):
        wid = lax.axis_index("s") * NC + lax.axis_index("c")
        pltpu.sync_copy(is_hbm.at[wid], is_v)
        pltpu.sync_copy(id_hbm.at[wid], id_v)
        base = wid * EW

        def body(j, carry):
            cs = pltpu.async_copy(xt_hbm.at[is_v.at[j]], rs_v, sem_s)
            cd = pltpu.async_copy(xt_hbm.at[id_v.at[j]], rd_v, sem_d)
            cs.wait()
            cd.wait()
            pltpu.sync_copy(rs_v, os_hbm.at[pl.ds(base + j * C, C)])
            pltpu.sync_copy(rd_v, od_hbm.at[pl.ds(base + j * C, C)])
            return carry

        lax.fori_loop(0, NCH, body, 0)

    return k(xt, idx_src3, idx_dst3)


def _cagat_layer(h, s3_aux, W_r, rel_emb, Wl, bl):
    rel_p, dst_p, src3, dst3, idx_src3, idx_dst3 = s3_aux
    xt = _transform(h, W_r).reshape(R * N, D)
    hs_p, hd_p = _sc_gather2(xt, idx_src3, idx_dst3)
    s3, v3, bidx3 = _scores(hs_p, hd_p, rel_p, dst_p, rel_emb)
    A2 = _sc_banded_denom(v3, bidx3)
    lse = _lse(A2)
    S2 = _sc_weighted_agg(s3, src3, dst3, lse, h)
    return _linear_relu(S2, Wl, bl)


def kernel(x, edge_index, batch, edge_attr, W_r, relation_embedding,
           bn0_w, bn0_b, bn1_w, bn1_b, Wl0, bl0, Wl1, bl1,
           bnh_w, bnh_b, pW1, pb1, pW2, pb2):
    src = edge_index[0]
    dst = edge_index[1]
    rel = edge_attr

    def padf(a):
        return jnp.concatenate([a, jnp.zeros((E_PAD - E,), a.dtype)])

    def pad3(a):
        return padf(a).reshape(NW, NCH, C)

    src_p = padf(src)
    dst_p = padf(dst)
    rel_p = padf(rel)
    s3_aux = (rel_p, dst_p, src_p.reshape(NW, NCH, C),
              dst_p.reshape(NW, NCH, C), pad3(rel * N + src),
              pad3(rel * N + dst))

    h0 = _batchnorm(x, bn0_w, bn0_b)
    h1 = _cagat_layer(h0, s3_aux, W_r, relation_embedding, Wl0, bl0)
    h2in = _batchnorm(h1, bn1_w, bn1_b)
    h2 = _cagat_layer(h2in, s3_aux, W_r, relation_embedding, Wl1, bl1)

    cat = jnp.concatenate([x, h1, h2], axis=1)
    return _pool_head(cat, batch, bnh_w, bnh_b, pW1, pb1, pW2, pb2)


# speedup vs baseline: 2.4973x; 1.0119x over previous
import functools

import jax
import jax.numpy as jnp
from jax import lax
from jax.experimental import pallas as pl
from jax.experimental.pallas import tpu as pltpu
from jax.experimental.pallas import tpu_sc as plsc

N = 10000
E = 160000
D = 128
R = 16
G = 256
H = 3 * D

NC = 2
NS = 16
NW = NC * NS
C = 128
E_PAD = 163840
EW = E_PAD // NW
NCH = EW // C
N_PAD = 10240


def _bn_body(x_ref, w_ref, b_ref, o_ref):
    x = x_ref[...]
    mu = jnp.mean(x, axis=0, keepdims=True)
    var = jnp.mean(jnp.square(x - mu), axis=0, keepdims=True)
    o_ref[...] = (x - mu) * jax.lax.rsqrt(var + 1e-5) * w_ref[...] + b_ref[...]


def _batchnorm(x, w, b):
    n, d = x.shape
    return pl.pallas_call(
        _bn_body,
        out_shape=jax.ShapeDtypeStruct((n, d), jnp.float32),
    )(x, w.reshape(1, d), b.reshape(1, d))


def _transform_body(h_ref, w_ref, o_ref):
    o_ref[0, ...] = jax.lax.dot_general(
        h_ref[...], w_ref[0, ...], (((1,), (1,)), ((), ())),
        preferred_element_type=jnp.float32)


def _transform(h, W_r):
    NB = 10
    nb = N // NB
    return pl.pallas_call(
        _transform_body,
        grid=(R, NB),
        in_specs=[
            pl.BlockSpec((nb, D), lambda r, i: (i, 0)),
            pl.BlockSpec((1, D, D), lambda r, i: (r, 0, 0)),
        ],
        out_specs=pl.BlockSpec((1, nb, D), lambda r, i: (r, i, 0)),
        out_shape=jax.ShapeDtypeStruct((R, N, D), jnp.float32),
    )(h, W_r)


def _score_body(hs_ref, hd_ref, rel_ref, dst_ref, emb_ref,
                s_ref, v_ref, bidx_ref):
    rel = rel_ref[0, ...]
    onehot = (rel[:, None] == lax.broadcasted_iota(jnp.int32, (1, R), 1)
              ).astype(jnp.float32)
    e_r = jax.lax.dot_general(onehot, emb_ref[...], (((1,), (0,)), ((), ())),
                              preferred_element_type=jnp.float32,
                              precision=lax.Precision.HIGHEST)
    t = jnp.tanh(hs_ref[...] + e_r)
    s = jnp.sum(hd_ref[...] * t, axis=1)
    eb = s_ref.shape[1]
    i = pl.program_id(0)
    gidx = i * eb + lax.broadcasted_iota(jnp.int32, (1, eb), 1)[0]
    s = jnp.where(gidx < E, s, -1e30)
    s_ref[0, ...] = s
    k2 = s * 1.4426950408889634
    b = jnp.clip(jnp.floor(k2 * 0.03125), -8.0, 7.0)
    v_ref[0, ...] = jnp.exp2(k2 - 32.0 * b)
    bidx_ref[0, ...] = (b.astype(jnp.int32) + 8) * N + dst_ref[0, ...]


def _scores(h_src, h_dst, rel_p, dst_p, rel_emb):
    EB = 32
    eb = E_PAD // EB
    s, v, bidx = pl.pallas_call(
        _score_body,
        grid=(EB,),
        in_specs=[
            pl.BlockSpec((eb, D), lambda i: (i, 0)),
            pl.BlockSpec((eb, D), lambda i: (i, 0)),
            pl.BlockSpec((1, eb), lambda i: (0, i)),
            pl.BlockSpec((1, eb), lambda i: (0, i)),
            pl.BlockSpec((R, D), lambda i: (0, 0)),
        ],
        out_specs=[
            pl.BlockSpec((1, eb), lambda i: (0, i)),
            pl.BlockSpec((1, eb), lambda i: (0, i)),
            pl.BlockSpec((1, eb), lambda i: (0, i)),
        ],
        out_shape=[
            jax.ShapeDtypeStruct((1, E_PAD), jnp.float32),
            jax.ShapeDtypeStruct((1, E_PAD), jnp.float32),
            jax.ShapeDtypeStruct((1, E_PAD), jnp.int32),
        ],
    )(h_src, h_dst, rel_p.reshape(1, E_PAD), dst_p.reshape(1, E_PAD), rel_emb)
    return (s.reshape(NW, NCH, C), v.reshape(NW, NCH, C),
            bidx.reshape(NW, NCH, C))


NBAND = 16


def _sc_banded_denom(v3, bidx3):
    mesh = plsc.VectorSubcoreMesh(core_axis_name="c", subcore_axis_name="s")

    @functools.partial(
        pl.kernel, mesh=mesh,
        out_type=jax.ShapeDtypeStruct((NC * NBAND * N,), jnp.float32),
        scratch_types=[
            pltpu.VMEM((NCH, C), jnp.float32),
            pltpu.VMEM((NCH, C), jnp.int32),
            pltpu.VMEM((NBAND * N // NS,), jnp.float32),
            pltpu.VMEM_SHARED((NBAND * N,), jnp.float32),
        ],
    )
    def k(v_hbm, b_hbm, out_hbm, v_v, b_v, o_v, acc):
        cid = lax.axis_index("c")
        sid = lax.axis_index("s")
        wid = sid * NC + cid

        chunk = NBAND * N // NS

        def zbody(i, carry):
            o_v[pl.ds(i * 16, 16)] = jnp.zeros((16,), jnp.float32)
            return carry

        lax.fori_loop(0, chunk // 16, zbody, 0)
        pltpu.sync_copy(o_v, acc.at[pl.ds(sid * chunk, chunk)])
        plsc.subcore_barrier()

        pltpu.sync_copy(v_hbm.at[wid], v_v)
        pltpu.sync_copy(b_hbm.at[wid], b_v)

        def body(j, carry):
            pltpu.sync_copy(v_v.at[j], acc.at[b_v.at[j]], add=True)
            return carry

        lax.fori_loop(0, NCH, body, 0)
        plsc.subcore_barrier()
        pltpu.sync_copy(acc.at[pl.ds(sid * chunk, chunk)], o_v)
        pltpu.sync_copy(o_v, out_hbm.at[pl.ds(cid * (NBAND * N) + sid * chunk,
                                              chunk)])

    return k(v3, bidx3)


def _lse_body(a_ref, o_ref):
    at = a_ref[0, ...] + a_ref[1, ...]
    biota = lax.broadcasted_iota(jnp.int32, (NBAND, N), 0).astype(jnp.float32)
    btop = jnp.max(jnp.where(at > 0.0, biota, -1.0), axis=0)
    fac = jnp.exp2(32.0 * jnp.clip(biota - btop[None, :], -2.0, 0.0))
    denom = jnp.sum(at * fac, axis=0)
    lse = jnp.where(btop >= 0.0,
                    0.6931471805599453 * 32.0 * (btop - 8.0)
                    + jnp.log(jnp.maximum(denom, 1e-38)), 0.0)
    o_ref[0, ...] = lse


def _lse(A2):
    return pl.pallas_call(
        _lse_body,
        out_shape=jax.ShapeDtypeStruct((1, N), jnp.float32),
    )(A2.reshape(NC, NBAND, N)).reshape(N)


def _sc_weighted_agg(s3, src3, dst3, lse, h):
    mesh = plsc.VectorSubcoreMesh(core_axis_name="c", subcore_axis_name="s")
    ROWZ = 128

    @functools.partial(
        pl.kernel, mesh=mesh,
        compiler_params=pltpu.CompilerParams(needs_layout_passes=False),
        out_type=jax.ShapeDtypeStruct((NC, N_PAD, D), jnp.float32),
        scratch_types=[
            pltpu.VMEM((C,), jnp.float32),
            pltpu.VMEM((C,), jnp.int32),
            pltpu.VMEM((C,), jnp.int32),
            pltpu.VMEM((N,), jnp.float32),
            pltpu.VMEM((C, D), jnp.float32),
            pltpu.VMEM((C,), jnp.float32),
            pltpu.VMEM_SHARED((N_PAD, D), jnp.float32),
            pltpu.SemaphoreType.DMA,
        ],
    )
    def k(s_hbm, src_hbm, dst_hbm, lse_hbm, h_hbm, out_hbm,
          sc_v, srcc_v, dstc_v, lse_v, rows_v, w_v, acc, sem):
        cid = lax.axis_index("c")
        sid = lax.axis_index("s")
        wid = sid * NC + cid

        def zbody(i, carry):
            for kk in range(D // 16):
                rows_v[i, pl.ds(kk * 16, 16)] = jnp.zeros((16,), jnp.float32)
            return carry

        lax.fori_loop(0, ROWZ, zbody, 0)
        for t in range(5):
            pltpu.sync_copy(
                rows_v, acc.at[pl.ds(sid * (N_PAD // NS) + t * ROWZ, ROWZ)])
        plsc.subcore_barrier()

        pltpu.sync_copy(lse_hbm, lse_v)

        def body(j, carry):
            pltpu.sync_copy(s_hbm.at[wid, j], sc_v)
            pltpu.sync_copy(src_hbm.at[wid, j], srcc_v)
            pltpu.sync_copy(dst_hbm.at[wid, j], dstc_v)
            pltpu.async_copy(h_hbm.at[srcc_v], rows_v, sem).wait()
            for i in range(C // 16):
                dvec = dstc_v[pl.ds(i * 16, 16)]
                lg = plsc.load_gather(lse_v, [dvec])
                w_v[pl.ds(i * 16, 16)] = jnp.exp(sc_v[pl.ds(i * 16, 16)] - lg)

            def scale(e, carry2):
                wb = plsc.load_gather(w_v, [jnp.full((16,), e, jnp.int32)])
                row = rows_v.at[e]
                for kk in range(D // 16):
                    row[pl.ds(kk * 16, 16)] = row[pl.ds(kk * 16, 16)] * wb
                return carry2

            lax.fori_loop(0, C, scale, 0)
            pltpu.sync_copy(rows_v, acc.at[dstc_v], add=True)
            return carry

        lax.fori_loop(0, NCH, body, 0)
        plsc.subcore_barrier()
        for t in range(5):
            r0 = sid * (N_PAD // NS) + t * ROWZ
            pltpu.sync_copy(acc.at[pl.ds(r0, ROWZ)], rows_v)
            pltpu.sync_copy(rows_v, out_hbm.at[cid, pl.ds(r0, ROWZ)])

    return k(s3, src3, dst3, lse, h)


def _linear_body(sa_ref, sb_ref, w_ref, b_ref, o_ref):
    agg = sa_ref[0, ...] + sb_ref[0, ...]
    y = jax.lax.dot_general(agg, w_ref[...], (((1,), (1,)), ((), ())),
                            preferred_element_type=jnp.float32)
    o_ref[...] = jax.nn.relu(y + b_ref[...])


def _linear_relu(S2, Wl, bl):
    NB = 10
    nb = N // NB
    return pl.pallas_call(
        _linear_body,
        grid=(NB,),
        in_specs=[
            pl.BlockSpec((1, nb, D), lambda i: (0, i, 0)),
            pl.BlockSpec((1, nb, D), lambda i: (1, i, 0)),
            pl.BlockSpec((D, D), lambda i: (0, 0)),
            pl.BlockSpec((1, D), lambda i: (0, 0)),
        ],
        out_specs=pl.BlockSpec((nb, D), lambda i: (i, 0)),
        out_shape=jax.ShapeDtypeStruct((N, D), jnp.float32),
    )(S2, S2, Wl, bl.reshape(1, D))


def _head_body(cat_ref, batch_ref, w_ref, b_ref, pw1_ref, pb1_ref,
               pw2_ref, pb2_ref, o_ref):
    onehot = (batch_ref[0, :][:, None] ==
              lax.broadcasted_iota(jnp.int32, (1, G), 1)).astype(jnp.float32)
    sums = jax.lax.dot_general(onehot, cat_ref[...], (((0,), (0,)), ((), ())),
                               preferred_element_type=jnp.float32, precision=lax.Precision.HIGHEST)
    counts = jnp.sum(onehot, axis=0)
    pooled = sums / jnp.maximum(counts, 1.0)[:, None]
    mu = jnp.mean(pooled, axis=0, keepdims=True)
    var = jnp.mean(jnp.square(pooled - mu), axis=0, keepdims=True)
    pooled = (pooled - mu) * jax.lax.rsqrt(var + 1e-5) * w_ref[...] + b_ref[...]
    y = jax.nn.relu(
        jax.lax.dot_general(pooled, pw1_ref[...], (((1,), (1,)), ((), ())),
                            preferred_element_type=jnp.float32) + pb1_ref[...])
    o_ref[...] = jax.lax.dot_general(
        y, pw2_ref[...], (((1,), (1,)), ((), ())),
        preferred_element_type=jnp.float32) + pb2_ref[...]


def _pool_head(cat, batch, bnh_w, bnh_b, pW1, pb1, pW2, pb2):
    return pl.pallas_call(
        _head_body,
        out_shape=jax.ShapeDtypeStruct((G, H), jnp.float32),
    )(cat, batch.reshape(1, N), bnh_w.reshape(1, H), bnh_b.reshape(1, H),
      pW1, pb1.reshape(1, H), pW2, pb2.reshape(1, H))


def _sc_gather2(xt, idx_src3, idx_dst3):
    mesh = plsc.VectorSubcoreMesh(core_axis_name="c", subcore_axis_name="s")

    @functools.partial(
        pl.kernel, mesh=mesh,
        out_type=[jax.ShapeDtypeStruct((E_PAD, D), jnp.float32),
                  jax.ShapeDtypeStruct((E_PAD, D), jnp.float32)],
        scratch_types=[
            pltpu.VMEM((NCH, C), jnp.int32),
            pltpu.VMEM((NCH, C), jnp.int32),
            pltpu.VMEM((2, C, D), jnp.float32),
            pltpu.VMEM((2, C, D), jnp.float32),
            pltpu.SemaphoreType.DMA,
            pltpu.SemaphoreType.DMA,
            pltpu.SemaphoreType.DMA,
            pltpu.SemaphoreType.DMA,
        ],
    )
    def k(xt_hbm, is_hbm, id_hbm, os_hbm, od_hbm,
          is_v, id_v, rs_v, rd_v, sem_s0, sem_d0, sem_s1, sem_d1):
        wid = lax.axis_index("s") * NC + lax.axis_index("c")
        pltpu.sync_copy(is_hbm.at[wid], is_v)
        pltpu.sync_copy(id_hbm.at[wid], id_v)
        base = wid * EW

        def body(p, carry):
            j0 = 2 * p
            j1 = 2 * p + 1
            c0 = pltpu.async_copy(xt_hbm.at[is_v.at[j0]], rs_v.at[0], sem_s0)
            d0 = pltpu.async_copy(xt_hbm.at[id_v.at[j0]], rd_v.at[0], sem_d0)
            c1 = pltpu.async_copy(xt_hbm.at[is_v.at[j1]], rs_v.at[1], sem_s1)
            d1 = pltpu.async_copy(xt_hbm.at[id_v.at[j1]], rd_v.at[1], sem_d1)
            c0.wait()
            d0.wait()
            pltpu.sync_copy(rs_v.at[0], os_hbm.at[pl.ds(base + j0 * C, C)])
            pltpu.sync_copy(rd_v.at[0], od_hbm.at[pl.ds(base + j0 * C, C)])
            c1.wait()
            d1.wait()
            pltpu.sync_copy(rs_v.at[1], os_hbm.at[pl.ds(base + j1 * C, C)])
            pltpu.sync_copy(rd_v.at[1], od_hbm.at[pl.ds(base + j1 * C, C)])
            return carry

        lax.fori_loop(0, NCH // 2, body, 0)

    return k(xt, idx_src3, idx_dst3)


def _cagat_layer(h, s3_aux, W_r, rel_emb, Wl, bl):
    rel_p, dst_p, src3, dst3, idx_src3, idx_dst3 = s3_aux
    xt = _transform(h, W_r).reshape(R * N, D)
    hs_p, hd_p = _sc_gather2(xt, idx_src3, idx_dst3)
    s3, v3, bidx3 = _scores(hs_p, hd_p, rel_p, dst_p, rel_emb)
    A2 = _sc_banded_denom(v3, bidx3)
    lse = _lse(A2)
    S2 = _sc_weighted_agg(s3, src3, dst3, lse, h)
    return _linear_relu(S2, Wl, bl)


def kernel(x, edge_index, batch, edge_attr, W_r, relation_embedding,
           bn0_w, bn0_b, bn1_w, bn1_b, Wl0, bl0, Wl1, bl1,
           bnh_w, bnh_b, pW1, pb1, pW2, pb2):
    src = edge_index[0]
    dst = edge_index[1]
    rel = edge_attr

    def padf(a):
        return jnp.concatenate([a, jnp.zeros((E_PAD - E,), a.dtype)])

    def pad3(a):
        return padf(a).reshape(NW, NCH, C)

    src_p = padf(src)
    dst_p = padf(dst)
    rel_p = padf(rel)
    s3_aux = (rel_p, dst_p, src_p.reshape(NW, NCH, C),
              dst_p.reshape(NW, NCH, C), pad3(rel * N + src),
              pad3(rel * N + dst))

    h0 = _batchnorm(x, bn0_w, bn0_b)
    h1 = _cagat_layer(h0, s3_aux, W_r, relation_embedding, Wl0, bl0)
    h2in = _batchnorm(h1, bn1_w, bn1_b)
    h2 = _cagat_layer(h2in, s3_aux, W_r, relation_embedding, Wl1, bl1)

    cat = jnp.concatenate([x, h1, h2], axis=1)
    return _pool_head(cat, batch, bnh_w, bnh_b, pW1, pb1, pW2, pb2)
